# baseline (speedup 1.0000x reference)
import functools

import jax
import jax.numpy as jnp
from jax import lax
from jax.experimental import pallas as pl
from jax.experimental.pallas import tpu as pltpu
from jax.experimental.pallas import tpu_sc as plsc

BATCH = 16384
EMB = 64
CAT_EMB = 32

_NC = 2
_NS = 16
_NW = _NC * _NS
_B_PER_W = BATCH // _NW

_sc_mesh = plsc.VectorSubcoreMesh(core_axis_name="c", subcore_axis_name="s")


@functools.partial(
    pl.kernel,
    out_type=[
        jax.ShapeDtypeStruct((BATCH, EMB), jnp.float32),
        jax.ShapeDtypeStruct((BATCH, CAT_EMB), jnp.float32),
    ],
    mesh=_sc_mesh,
    scratch_types=[
        pltpu.VMEM((_B_PER_W,), jnp.int32),
        pltpu.VMEM((_B_PER_W,), jnp.int32),
        pltpu.SemaphoreType.DMA,
    ],
)
def _sc_gather(ids_hbm, cids_hbm, item_table_hbm, cat_table_hbm,
               item_out, cat_out, idx_v, cidx_v, sem):
    wid = lax.axis_index("s") * _NC + lax.axis_index("c")
    base = wid * _B_PER_W
    pltpu.sync_copy(ids_hbm.at[pl.ds(base, _B_PER_W)], idx_v)
    pltpu.sync_copy(cids_hbm.at[pl.ds(base, _B_PER_W)], cidx_v)

    def body(g, carry):
        iv = idx_v[pl.ds(g * 16, 16)]
        cv = cidx_v[pl.ds(g * 16, 16)]
        for l in range(16):
            j = g * 16 + l
            pltpu.async_copy(item_table_hbm.at[pl.ds(iv[l], 1)],
                             item_out.at[pl.ds(base + j, 1)], sem)
            pltpu.async_copy(cat_table_hbm.at[pl.ds(cv[l], 1)],
                             cat_out.at[pl.ds(base + j, 1)], sem)
        return carry

    lax.fori_loop(0, _B_PER_W // 16, body, 0)
    pltpu.make_async_copy(item_table_hbm.at[pl.ds(0, _B_PER_W)],
                          item_out.at[pl.ds(base, _B_PER_W)], sem).wait()
    pltpu.make_async_copy(cat_table_hbm.at[pl.ds(0, _B_PER_W)],
                          cat_out.at[pl.ds(base, _B_PER_W)], sem).wait()


_BM = 2048


def _dense_body(x1_ref, x2_ref, w1_ref, w2_ref, b_ref, o_ref):
    o_ref[...] = (
        jnp.dot(x1_ref[...], w1_ref[...], preferred_element_type=jnp.float32)
        + jnp.dot(x2_ref[...], w2_ref[...], preferred_element_type=jnp.float32)
        + b_ref[...]
    )


_tc_dense = pl.pallas_call(
    _dense_body,
    grid=(BATCH // _BM,),
    in_specs=[
        pl.BlockSpec((_BM, EMB), lambda i: (i, 0)),
        pl.BlockSpec((_BM, CAT_EMB), lambda i: (i, 0)),
        pl.BlockSpec((EMB, EMB), lambda i: (0, 0)),
        pl.BlockSpec((CAT_EMB, EMB), lambda i: (0, 0)),
        pl.BlockSpec((1, EMB), lambda i: (0, 0)),
    ],
    out_specs=pl.BlockSpec((_BM, EMB), lambda i: (i, 0)),
    out_shape=jax.ShapeDtypeStruct((BATCH, EMB), jnp.float32),
)


def kernel(item_id, item_category, item_table, cat_table, W, b):
    item_emb, cat_emb = _sc_gather(item_id, item_category, item_table,
                                   cat_table)
    return _tc_dense(item_emb, cat_emb, W[:EMB], W[EMB:], b.reshape(1, EMB))


# speedup vs baseline: 1.2960x; 1.2960x over previous
import functools

import jax
import jax.numpy as jnp
from jax import lax
from jax.experimental import pallas as pl
from jax.experimental.pallas import tpu as pltpu
from jax.experimental.pallas import tpu_sc as plsc

BATCH = 16384
EMB = 64
CAT_EMB = 32
LANES = 128

_NC = 2
_NS = 16
_NW = _NC * _NS
_CHUNK = 128
_B_PER_W = BATCH // _NW
_ROUND = 256

_sc_mesh = plsc.VectorSubcoreMesh(core_axis_name="c", subcore_axis_name="s")


@functools.partial(
    pl.kernel,
    out_type=[
        jax.ShapeDtypeStruct((BATCH, LANES), jnp.float32),
        jax.ShapeDtypeStruct((BATCH, LANES), jnp.float32),
    ],
    mesh=_sc_mesh,
    scratch_types=[
        pltpu.VMEM((_B_PER_W,), jnp.int32),
        pltpu.VMEM((_B_PER_W,), jnp.int32),
        pltpu.VMEM((_ROUND, LANES), jnp.float32),
        pltpu.VMEM((_ROUND, LANES), jnp.float32),
        pltpu.SemaphoreType.DMA,
    ],
)
def _sc_gather(ids_hbm, cids_hbm, itemp_hbm, catp_hbm,
               item_out, cat_out, idx_v, cidx_v, buf_a, buf_b, sem):
    wid = lax.axis_index("s") * _NC + lax.axis_index("c")
    base = wid * _B_PER_W
    pltpu.sync_copy(ids_hbm.at[pl.ds(base, _B_PER_W)], idx_v)
    pltpu.sync_copy(cids_hbm.at[pl.ds(base, _B_PER_W)], cidx_v)
    bufs = (buf_a, buf_b)
    for r in range(_B_PER_W // _ROUND):
        buf = bufs[r % 2]
        cps = [
            pltpu.async_copy(
                itemp_hbm.at[idx_v.at[pl.ds(r * _ROUND + k * _CHUNK, _CHUNK)]],
                buf.at[pl.ds(k * _CHUNK, _CHUNK)], sem)
            for k in range(_ROUND // _CHUNK)
        ]
        for cp in cps:
            cp.wait()
        pltpu.sync_copy(buf, item_out.at[pl.ds(base + r * _ROUND, _ROUND)])
    for r in range(_B_PER_W // _ROUND):
        buf = bufs[r % 2]
        cps = [
            pltpu.async_copy(
                catp_hbm.at[cidx_v.at[pl.ds(r * _ROUND + k * _CHUNK, _CHUNK)]],
                buf.at[pl.ds(k * _CHUNK, _CHUNK)], sem)
            for k in range(_ROUND // _CHUNK)
        ]
        for cp in cps:
            cp.wait()
        pltpu.sync_copy(buf, cat_out.at[pl.ds(base + r * _ROUND, _ROUND)])


_BM = 2048


def _dense_body(ip_ref, cp_ref, pi_ref, oh_ref, w1_ref, w2_ref, b_ref,
                o_ref):
    ip = ip_ref[...]
    cp = cp_ref[...]
    pi = pi_ref[...]
    oh = oh_ref[...]
    xi = ip[:, :EMB] * (1.0 - pi) + ip[:, EMB:] * pi
    xc = jnp.zeros((_BM, CAT_EMB), jnp.float32)
    for q in range(4):
        xc = xc + cp[:, q * CAT_EMB:(q + 1) * CAT_EMB] * oh[:, q][:, None]
    o_ref[...] = (
        jnp.dot(xi, w1_ref[...], preferred_element_type=jnp.float32)
        + jnp.dot(xc, w2_ref[...], preferred_element_type=jnp.float32)
        + b_ref[...]
    )


_tc_dense = pl.pallas_call(
    _dense_body,
    grid=(BATCH // _BM,),
    in_specs=[
        pl.BlockSpec((_BM, LANES), lambda i: (i, 0)),
        pl.BlockSpec((_BM, LANES), lambda i: (i, 0)),
        pl.BlockSpec((_BM, 1), lambda i: (i, 0)),
        pl.BlockSpec((_BM, 4), lambda i: (i, 0)),
        pl.BlockSpec((EMB, EMB), lambda i: (0, 0)),
        pl.BlockSpec((CAT_EMB, EMB), lambda i: (0, 0)),
        pl.BlockSpec((1, EMB), lambda i: (0, 0)),
    ],
    out_specs=pl.BlockSpec((_BM, EMB), lambda i: (i, 0)),
    out_shape=jax.ShapeDtypeStruct((BATCH, EMB), jnp.float32),
)


def kernel(item_id, item_category, item_table, cat_table, W, b):
    itemp = item_table.reshape(item_table.shape[0] // 2, 2 * EMB)
    catp = cat_table.reshape(cat_table.shape[0] // 4, 4 * CAT_EMB)
    ids_half = item_id >> 1
    cids_quarter = item_category >> 2
    par_i = (item_id & 1).astype(jnp.float32)[:, None]
    oh_c = ((item_category & 3)[:, None]
            == jnp.arange(4, dtype=jnp.int32)).astype(jnp.float32)
    ipair, cpair = _sc_gather(ids_half, cids_quarter, itemp, catp)
    return _tc_dense(ipair, cpair, par_i, oh_c, W[:EMB], W[EMB:],
                     b.reshape(1, EMB))
